# Initial kernel scaffold; baseline (speedup 1.0000x reference)
import jax
import jax.numpy as jnp
from jax.experimental import pallas as pl


def kernel(x, edge_index, W1, b1, W2, b2):
    raise NotImplementedError("write your pallas kernel here")


# speedup vs baseline: 8.7160x; 8.7160x over previous
import functools

import jax
import jax.numpy as jnp
from jax import lax
from jax.experimental import pallas as pl
from jax.experimental.pallas import tpu as pltpu
from jax.experimental.pallas import tpu_sc as plsc

N_NODES = 10000
N_EDGES = 320000
D_IN = 128
D_HID = 128
D_OUT = 768

NC = 2
NS = 16
NW = NC * NS
HN = N_NODES // NC
HA = 5120
IB = 128
RPT = 80
E_PAD = NS * RPT * IB
DEG_W = 16
NBUF = 4
RPT_OUT = HA // NS

RPT_ALL = 160
E_ALL = NS * RPT_ALL * IB


@functools.cache
def _sc_kernels():
    mesh = plsc.VectorSubcoreMesh(
        core_axis_name="c", subcore_axis_name="s", num_cores=NC, num_subcores=NS
    )

    @functools.partial(
        pl.kernel,
        out_type=jax.ShapeDtypeStruct((NW, HA), jnp.float32),
        mesh=mesh,
        scratch_types=[
            pltpu.VMEM((8, IB), jnp.int32),
            pltpu.VMEM((HA,), jnp.float32),
        ],
        compiler_params=pltpu.CompilerParams(needs_layout_passes=False),
    )
    def deg_pass(dst_hbm, zeros_hbm, out_hbm, idx_v, hist_v):
        c = lax.axis_index("c")
        s = lax.axis_index("s")
        wid = c * NS + s
        lo = c * HN
        pltpu.sync_copy(zeros_hbm, hist_v)

        def chunk(q, carry):
            pltpu.sync_copy(
                dst_hbm.at[pl.ds(s * RPT_ALL + q * 8, 8)], idx_v)
            for j in range(8):
                for k in range(IB // 16):
                    d = idx_v[j, pl.ds(k * 16, 16)]
                    t = d - lo
                    ok = (t >= 0) & (t < HN)
                    row = jnp.where(ok, t, HN + (d & 63))
                    cnt, last = plsc.scan_count(row)
                    plsc.addupdate_scatter(
                        hist_v, [row], cnt.astype(jnp.float32), mask=last)
            return carry

        lax.fori_loop(0, RPT_ALL // 8, chunk, 0)
        pltpu.sync_copy(hist_v, out_hbm.at[wid])

    @functools.partial(
        pl.kernel,
        out_type=jax.ShapeDtypeStruct((NC, HA, D_HID), jnp.float32),
        mesh=mesh,
        scratch_types=[
            pltpu.VMEM((16, IB), jnp.int32),
            pltpu.VMEM((16, IB), jnp.int32),
            pltpu.VMEM((IB, D_HID), jnp.float32),
            pltpu.SemaphoreType.DMA,
            pltpu.VMEM_SHARED((HA, D_HID), jnp.float32),
        ],
        compiler_params=pltpu.CompilerParams(needs_layout_passes=False),
    )
    def agg_pass(g_hbm, src_hbm, dst_hbm, zeros_hbm, out_hbm,
                 src_v, dst_v, rows_v, sem, acc_sh):
        c = lax.axis_index("c")
        s = lax.axis_index("s")
        lo = c * HN
        pltpu.sync_copy(zeros_hbm, acc_sh.at[pl.ds(s * RPT_OUT, RPT_OUT)])
        plsc.subcore_barrier()

        def chunk(q, carry):
            base = s * RPT_ALL + q * 16
            pltpu.sync_copy(src_hbm.at[pl.ds(base, 16)], src_v)
            pltpu.sync_copy(dst_hbm.at[pl.ds(base, 16)], dst_v)
            for j in range(16):
                for k in range(IB // 16):
                    sl = pl.ds(k * 16, 16)
                    d = dst_v[j, sl]
                    t = d - lo
                    ok = (t >= 0) & (t < HN)
                    dst_v[j, sl] = jnp.where(ok, t, HN + (d & 63))

            def row(j, carry2):
                pltpu.async_copy(g_hbm.at[src_v.at[j]], rows_v, sem).wait()
                pltpu.sync_copy(rows_v, acc_sh.at[dst_v.at[j]], add=True)
                return carry2

            lax.fori_loop(0, 16, row, 0)
            return carry

        lax.fori_loop(0, RPT_ALL // 16, chunk, 0)
        plsc.subcore_barrier()
        pltpu.sync_copy(
            acc_sh.at[pl.ds(s * RPT_OUT, RPT_OUT)],
            out_hbm.at[c].at[pl.ds(s * RPT_OUT, RPT_OUT)],
        )

    return deg_pass, agg_pass


_RB = 1000
_GRID = N_NODES // _RB
_BPH = HN // _RB


def _scale_body(x_ref, w_ref, degp_ref, og_ref, od_ref):
    ci = pl.program_id(0) // _BPH
    col = jax.lax.broadcasted_iota(jnp.int32, (_RB, NW), 1)
    part = jnp.where(col // NS == ci, degp_ref[...], 0.0)
    deg = jnp.sum(part, axis=1) + 1.0
    dinv = lax.rsqrt(deg)
    h = jnp.dot(x_ref[...], w_ref[...], precision=lax.Precision.HIGHEST)
    og_ref[...] = h * dinv[:, None]
    od_ref[...] = dinv[:, None]


def _mid_body(s_ref, g1_ref, dinv_ref, b1_ref, o_ref):
    dinv = dinv_ref[...]
    t = s_ref[0] + g1_ref[...]
    h = jnp.maximum(t * dinv + b1_ref[...], 0.0)
    o_ref[...] = h * dinv


def _out_body(s_ref, g2_ref, dinv_ref, w2_ref, b2_ref, o_ref):
    dinv = dinv_ref[...]
    a = (s_ref[0] + g2_ref[...]) * dinv
    o_ref[...] = (
        jnp.dot(a, w2_ref[...], precision=lax.Precision.HIGHEST) + b2_ref[...]
    )


def _row_spec(d):
    return pl.BlockSpec((_RB, d), lambda i: (i, 0))


def _half_spec(d):
    return pl.BlockSpec((1, _RB, d), lambda i: (i // _BPH, i % _BPH, 0))


_deg_spec = pl.BlockSpec((_RB, NW), lambda i: (i % _BPH, 0))
_vec_spec = pl.BlockSpec((_RB, 1), lambda i: (i, 0))


def _full_spec(shape):
    return pl.BlockSpec(shape, lambda i: tuple(0 for _ in shape))


_scale_call = pl.pallas_call(
    _scale_body,
    grid=(_GRID,),
    in_specs=[
        _row_spec(D_IN),
        _full_spec((D_IN, D_HID)),
        _deg_spec,
    ],
    out_specs=(_row_spec(D_HID), _vec_spec),
    out_shape=(
        jax.ShapeDtypeStruct((N_NODES, D_HID), jnp.float32),
        jax.ShapeDtypeStruct((N_NODES, 1), jnp.float32),
    ),
)

_mid_call = pl.pallas_call(
    _mid_body,
    grid=(_GRID,),
    in_specs=[
        _half_spec(D_HID),
        _row_spec(D_HID),
        _vec_spec,
        _full_spec((1, D_HID)),
    ],
    out_specs=_row_spec(D_HID),
    out_shape=jax.ShapeDtypeStruct((N_NODES, D_HID), jnp.float32),
)

_out_call = pl.pallas_call(
    _out_body,
    grid=(_GRID,),
    in_specs=[
        _half_spec(D_HID),
        _row_spec(D_HID),
        _vec_spec,
        _full_spec((D_HID, D_OUT)),
        _full_spec((1, D_OUT)),
    ],
    out_specs=_row_spec(D_OUT),
    out_shape=jax.ShapeDtypeStruct((N_NODES, D_OUT), jnp.float32),
)


def kernel(x, edge_index, W1, b1, W2, b2):
    src = edge_index[0].astype(jnp.int32)
    dst = edge_index[1].astype(jnp.int32)
    pad = E_ALL - N_EDGES
    pad_src = jnp.zeros((pad,), jnp.int32)
    pad_dst = jnp.full((pad,), N_NODES, jnp.int32)
    src2d = jnp.concatenate([src, pad_src]).reshape(NS * RPT_ALL, IB)
    dst2d = jnp.concatenate([dst, pad_dst]).reshape(NS * RPT_ALL, IB)
    zeros_deg = jnp.zeros((HA,), jnp.float32)
    zeros = jnp.zeros((RPT_OUT, D_HID), jnp.float32)

    deg_pass, agg_pass = _sc_kernels()
    degp = deg_pass(dst2d, zeros_deg).T
    g1, dinv = _scale_call(x, W1, degp)
    S1 = agg_pass(g1, src2d, dst2d, zeros)
    g2 = _mid_call(S1, g1, dinv, b1.reshape(1, D_HID))
    S2 = agg_pass(g2, src2d, dst2d, zeros)
    return _out_call(S2, g2, dinv, W2, b2.reshape(1, D_OUT))


# speedup vs baseline: 10.1169x; 1.1607x over previous
import functools

import jax
import jax.numpy as jnp
from jax import lax
from jax.experimental import pallas as pl
from jax.experimental.pallas import tpu as pltpu
from jax.experimental.pallas import tpu_sc as plsc

N_NODES = 10000
N_EDGES = 320000
D_IN = 128
D_HID = 128
D_OUT = 768

NC = 2
NS = 16
NW = NC * NS
HN = N_NODES // NC
IB = 128
RPT_ALL = 160
E_ALL = NS * RPT_ALL * IB
NA = 10240
N_PAD = NA - N_NODES
ROWS_PT = NA // NS
CH = 16
HA = HN + 120
RPT_DEG = RPT_ALL // 2


@functools.cache
def _sc_kernels():
    mesh = plsc.VectorSubcoreMesh(
        core_axis_name="c", subcore_axis_name="s", num_cores=NC, num_subcores=NS
    )

    @functools.partial(
        pl.kernel,
        out_type=jax.ShapeDtypeStruct((NW, NA), jnp.float32),
        mesh=mesh,
        scratch_types=[
            pltpu.VMEM((4, IB), jnp.int32),
            pltpu.VMEM((NA,), jnp.float32),
        ],
        compiler_params=pltpu.CompilerParams(needs_layout_passes=False),
    )
    def deg_pass(dst_hbm, zeros_hbm, out_hbm, idx_v, hist_v):
        c = lax.axis_index("c")
        s = lax.axis_index("s")
        wid = c * NS + s
        pltpu.sync_copy(zeros_hbm, hist_v)

        def chunk(q, carry):
            pltpu.sync_copy(dst_hbm.at[pl.ds(wid * RPT_DEG + q * 4, 4)], idx_v)
            for j in range(4):
                for k in range(IB // 16):
                    d = idx_v[j, pl.ds(k * 16, 16)]
                    cnt, last = plsc.scan_count(d)
                    plsc.addupdate_scatter(
                        hist_v, [d], cnt.astype(jnp.float32), mask=last)
            return carry

        lax.fori_loop(0, RPT_DEG // 4, chunk, 0)
        pltpu.sync_copy(hist_v, out_hbm.at[wid])

    @functools.partial(
        pl.kernel,
        out_type=(
            jax.ShapeDtypeStruct((NC, HA, D_HID), jnp.float32),
            jax.ShapeDtypeStruct((NC * NA, D_HID), jnp.float32),
        ),
        mesh=mesh,
        scratch_types=[
            pltpu.VMEM((CH, IB), jnp.int32),
            pltpu.VMEM((CH, IB), jnp.int32),
            pltpu.VMEM((IB, D_HID), jnp.float32),
            pltpu.VMEM((IB, D_HID), jnp.float32),
            pltpu.VMEM((D_HID,), jnp.float32),
            pltpu.SemaphoreType.DMA,
            pltpu.SemaphoreType.DMA,
            pltpu.VMEM_SHARED((NA, D_HID), jnp.float32),
        ],
        compiler_params=pltpu.CompilerParams(needs_layout_passes=False),
    )
    def gcn_pass(g1_hbm, src_hbm, dst_hbm, dinvb_hbm, b1_hbm, zeros_hbm,
                 s2_hbm, g2_hbm,
                 src_v, dst_v, r0, r1, b1_v,
                 s0, s1, acc_sh):
        c = lax.axis_index("c")
        s = lax.axis_index("s")
        rbase0 = s * ROWS_PT

        def sweep(tab_hbm, idx_off):
            def chunk(q, carry):
                base = s * RPT_ALL + q * CH
                pltpu.sync_copy(src_hbm.at[pl.ds(base, CH)], src_v)
                pltpu.sync_copy(dst_hbm.at[pl.ds(base, CH)], dst_v)
                if idx_off is not None:
                    for j in range(CH):
                        for k in range(IB // 16):
                            sl = pl.ds(k * 16, 16)
                            src_v[j, sl] = src_v[j, sl] + idx_off
                pltpu.async_copy(tab_hbm.at[src_v.at[0]], r0, s0)
                pltpu.async_copy(tab_hbm.at[src_v.at[1]], r1, s1)

                def pair(p, carry2):
                    for b, (rb, sb) in enumerate(((r0, s0), (r1, s1))):
                        j = p * 2 + b
                        pltpu.make_async_copy(
                            tab_hbm.at[src_v.at[j]], rb, sb).wait()
                        pltpu.sync_copy(rb, acc_sh.at[dst_v.at[j]], add=True)

                        @pl.when(p < CH // 2 - 1)
                        def _():
                            pltpu.async_copy(
                                tab_hbm.at[src_v.at[j + 2]], rb, sb)
                    return carry2

                lax.fori_loop(0, CH // 2, pair, 0)
                return carry

            lax.fori_loop(0, RPT_ALL // CH, chunk, 0)

        pltpu.sync_copy(zeros_hbm, acc_sh.at[pl.ds(rbase0, ROWS_PT)])
        plsc.subcore_barrier()
        sweep(g1_hbm, None)
        plsc.subcore_barrier()

        pltpu.sync_copy(b1_hbm, b1_v)
        TB = 32

        def tchunk(q, carry):
            rbase = rbase0 + q * TB
            pltpu.sync_copy(acc_sh.at[pl.ds(rbase, TB)], r0.at[pl.ds(0, TB)])
            pltpu.sync_copy(g1_hbm.at[pl.ds(rbase, TB)],
                            r0.at[pl.ds(TB, TB)])
            pltpu.sync_copy(dinvb_hbm.at[pl.ds(rbase, TB)],
                            r0.at[pl.ds(2 * TB, TB)])
            for jj in range(TB):
                for k in range(D_HID // 16):
                    sl = pl.ds(k * 16, 16)
                    dv = r0[2 * TB + jj, sl]
                    z = (r0[jj, sl] + r0[TB + jj, sl]) * dv + b1_v[sl]
                    r0[jj, sl] = jnp.maximum(z, 0.0) * dv
            pltpu.sync_copy(
                r0.at[pl.ds(0, TB)], g2_hbm.at[pl.ds(c * NA + rbase, TB)])
            return carry

        lax.fori_loop(0, ROWS_PT // TB, tchunk, 0)
        pltpu.sync_copy(zeros_hbm, acc_sh.at[pl.ds(rbase0, ROWS_PT)])
        plsc.subcore_barrier()

        sweep(g2_hbm, c * NA)
        plsc.subcore_barrier()

        pltpu.sync_copy(
            acc_sh.at[pl.ds(c * HN + s * (HA // NS), HA // NS)],
            s2_hbm.at[c].at[pl.ds(s * (HA // NS), HA // NS)],
        )

    return deg_pass, gcn_pass


_RB = 1000
_GRID = N_NODES // _RB
_BPH = HN // _RB


def _scale_body(x_ref, w_ref, degp_ref, og_ref, od_ref, ob_ref):
    deg = jnp.sum(degp_ref[...], axis=1) + 1.0
    dinv = lax.rsqrt(deg)
    h = jnp.dot(x_ref[...], w_ref[...], precision=lax.Precision.HIGHEST)
    og_ref[...] = h * dinv[:, None]
    od_ref[...] = dinv[:, None]
    ob_ref[...] = jnp.broadcast_to(dinv[:, None], ob_ref.shape)


def _out_body(s_ref, g2_ref, dinv_ref, w2_ref, b2_ref, o_ref):
    dinv = dinv_ref[...]
    a = (s_ref[0] + g2_ref[...]) * dinv
    o_ref[...] = (
        jnp.dot(a, w2_ref[...], precision=lax.Precision.HIGHEST) + b2_ref[...]
    )


def _row_spec(d):
    return pl.BlockSpec((_RB, d), lambda i: (i, 0))


def _half_spec(d):
    return pl.BlockSpec((1, _RB, d), lambda i: (i // _BPH, i % _BPH, 0))


_deg_spec = pl.BlockSpec((_RB, NW), lambda i: (i, 0))
_vec_spec = pl.BlockSpec((_RB, 1), lambda i: (i, 0))


def _full_spec(shape):
    return pl.BlockSpec(shape, lambda i: tuple(0 for _ in shape))


_scale_call = pl.pallas_call(
    _scale_body,
    grid=(_GRID,),
    in_specs=[
        _row_spec(D_IN),
        _full_spec((D_IN, D_HID)),
        _deg_spec,
    ],
    out_specs=(_row_spec(D_HID), _vec_spec, _row_spec(D_HID)),
    out_shape=(
        jax.ShapeDtypeStruct((NA, D_HID), jnp.float32),
        jax.ShapeDtypeStruct((NA, 1), jnp.float32),
        jax.ShapeDtypeStruct((NA, D_HID), jnp.float32),
    ),
)

_out_call = pl.pallas_call(
    _out_body,
    grid=(_GRID,),
    in_specs=[
        _half_spec(D_HID),
        _row_spec(D_HID),
        _vec_spec,
        _full_spec((D_HID, D_OUT)),
        _full_spec((1, D_OUT)),
    ],
    out_specs=_row_spec(D_OUT),
    out_shape=jax.ShapeDtypeStruct((N_NODES, D_OUT), jnp.float32),
)


def kernel(x, edge_index, W1, b1, W2, b2):
    src = edge_index[0].astype(jnp.int32)
    dst = edge_index[1].astype(jnp.int32)
    pad = E_ALL - N_EDGES
    pad_src = jnp.zeros((pad,), jnp.int32)
    pad_dst = N_NODES + (jnp.arange(pad, dtype=jnp.int32) % N_PAD)
    src2d = jnp.concatenate([src, pad_src]).reshape(NS * RPT_ALL, IB)
    dst2d = jnp.concatenate([dst, pad_dst]).reshape(NS * RPT_ALL, IB)
    zeros_deg = jnp.zeros((NA,), jnp.float32)
    zeros = jnp.zeros((ROWS_PT, D_HID), jnp.float32)

    deg_pass, gcn_pass = _sc_kernels()
    degp = deg_pass(dst2d, zeros_deg).T
    g1, dinv, dinvb = _scale_call(x, W1, degp)
    S2, g2 = gcn_pass(g1, src2d, dst2d, dinvb, b1, zeros)
    return _out_call(S2, g2, dinv, W2, b2.reshape(1, D_OUT))


# speedup vs baseline: 10.1783x; 1.0061x over previous
import functools

import jax
import jax.numpy as jnp
from jax import lax
from jax.experimental import pallas as pl
from jax.experimental.pallas import tpu as pltpu
from jax.experimental.pallas import tpu_sc as plsc

N_NODES = 10000
N_EDGES = 320000
D_IN = 128
D_HID = 128
D_OUT = 768

NC = 2
NS = 16
NW = NC * NS
HN = N_NODES // NC
IB = 128
RPT_ALL = 160
E_ALL = NS * RPT_ALL * IB
NA = 10240
N_PAD = NA - N_NODES
ROWS_PT = NA // NS
CH = 16
HA = HN + 120
RPT_DEG = RPT_ALL // 2


@functools.cache
def _sc_kernels():
    mesh = plsc.VectorSubcoreMesh(
        core_axis_name="c", subcore_axis_name="s", num_cores=NC, num_subcores=NS
    )

    @functools.partial(
        pl.kernel,
        out_type=jax.ShapeDtypeStruct((NW, NA), jnp.float32),
        mesh=mesh,
        scratch_types=[
            pltpu.VMEM((4, IB), jnp.int32),
            pltpu.VMEM((NA,), jnp.float32),
        ],
        compiler_params=pltpu.CompilerParams(needs_layout_passes=False),
    )
    def deg_pass(dst_hbm, zeros_hbm, out_hbm, idx_v, hist_v):
        c = lax.axis_index("c")
        s = lax.axis_index("s")
        wid = c * NS + s
        pltpu.sync_copy(zeros_hbm, hist_v)

        def chunk(q, carry):
            pltpu.sync_copy(dst_hbm.at[pl.ds(wid * RPT_DEG + q * 4, 4)], idx_v)
            for j in range(4):
                for k in range(IB // 16):
                    d = idx_v[j, pl.ds(k * 16, 16)]
                    cnt, last = plsc.scan_count(d)
                    plsc.addupdate_scatter(
                        hist_v, [d], cnt.astype(jnp.float32), mask=last)
            return carry

        lax.fori_loop(0, RPT_DEG // 4, chunk, 0)
        pltpu.sync_copy(hist_v, out_hbm.at[wid])

    @functools.partial(
        pl.kernel,
        out_type=(
            jax.ShapeDtypeStruct((NC, HA, D_HID), jnp.float32),
            jax.ShapeDtypeStruct((NC * NA, D_HID), jnp.float32),
        ),
        mesh=mesh,
        scratch_types=[
            pltpu.VMEM((CH, IB), jnp.int32),
            pltpu.VMEM((CH, IB), jnp.int32),
            pltpu.VMEM((IB, D_HID), jnp.float32),
            pltpu.VMEM((IB, D_HID), jnp.float32),
            pltpu.VMEM((D_HID,), jnp.float32),
            pltpu.SemaphoreType.DMA,
            pltpu.SemaphoreType.DMA,
            pltpu.SemaphoreType.DMA,
            pltpu.SemaphoreType.DMA,
            pltpu.VMEM_SHARED((NA, D_HID), jnp.float32),
        ],
        compiler_params=pltpu.CompilerParams(needs_layout_passes=False),
    )
    def gcn_pass(g1_hbm, src_hbm, dst_hbm, dinvb_hbm, b1_hbm, zeros_hbm,
                 s2_hbm, g2_hbm,
                 src_v, dst_v, r0, r1, b1_v,
                 s0, s1, t0, t1, acc_sh):
        c = lax.axis_index("c")
        s = lax.axis_index("s")
        rbase0 = s * ROWS_PT

        def sweep(tab_hbm, idx_off):
            def chunk(q, carry):
                base = s * RPT_ALL + q * CH
                pltpu.sync_copy(src_hbm.at[pl.ds(base, CH)], src_v)
                pltpu.sync_copy(dst_hbm.at[pl.ds(base, CH)], dst_v)
                if idx_off is not None:
                    for j in range(CH):
                        for k in range(IB // 16):
                            sl = pl.ds(k * 16, 16)
                            src_v[j, sl] = src_v[j, sl] + idx_off

                @pl.when(q > 0)
                def _():
                    pltpu.make_async_copy(
                        r0, acc_sh.at[dst_v.at[0]], t0).wait()
                    pltpu.make_async_copy(
                        r1, acc_sh.at[dst_v.at[0]], t1).wait()

                pltpu.async_copy(tab_hbm.at[src_v.at[0]], r0, s0)
                pltpu.async_copy(tab_hbm.at[src_v.at[1]], r1, s1)

                def pair(p, carry2):
                    for b, (rb, sb, tb) in enumerate(
                            ((r0, s0, t0), (r1, s1, t1))):
                        j = p * 2 + b
                        pltpu.make_async_copy(
                            tab_hbm.at[src_v.at[j]], rb, sb).wait()
                        pltpu.async_copy(
                            rb, acc_sh.at[dst_v.at[j]], tb, add=True)

                        @pl.when(p < CH // 2 - 1)
                        def _():
                            pltpu.make_async_copy(
                                rb, acc_sh.at[dst_v.at[j]], tb).wait()
                            pltpu.async_copy(
                                tab_hbm.at[src_v.at[j + 2]], rb, sb)
                    return carry2

                lax.fori_loop(0, CH // 2, pair, 0)
                return carry

            lax.fori_loop(0, RPT_ALL // CH, chunk, 0)
            pltpu.make_async_copy(r0, acc_sh.at[dst_v.at[0]], t0).wait()
            pltpu.make_async_copy(r1, acc_sh.at[dst_v.at[1]], t1).wait()

        pltpu.sync_copy(zeros_hbm, acc_sh.at[pl.ds(rbase0, ROWS_PT)])
        plsc.subcore_barrier()
        sweep(g1_hbm, None)
        plsc.subcore_barrier()

        pltpu.sync_copy(b1_hbm, b1_v)
        TB = 32

        def tchunk(q, carry):
            rbase = rbase0 + q * TB
            pltpu.sync_copy(acc_sh.at[pl.ds(rbase, TB)], r0.at[pl.ds(0, TB)])
            pltpu.sync_copy(g1_hbm.at[pl.ds(rbase, TB)],
                            r0.at[pl.ds(TB, TB)])
            pltpu.sync_copy(dinvb_hbm.at[pl.ds(rbase, TB)],
                            r0.at[pl.ds(2 * TB, TB)])
            for jj in range(TB):
                for k in range(D_HID // 16):
                    sl = pl.ds(k * 16, 16)
                    dv = r0[2 * TB + jj, sl]
                    z = (r0[jj, sl] + r0[TB + jj, sl]) * dv + b1_v[sl]
                    r0[jj, sl] = jnp.maximum(z, 0.0) * dv
            pltpu.sync_copy(
                r0.at[pl.ds(0, TB)], g2_hbm.at[pl.ds(c * NA + rbase, TB)])
            return carry

        lax.fori_loop(0, ROWS_PT // TB, tchunk, 0)
        pltpu.sync_copy(zeros_hbm, acc_sh.at[pl.ds(rbase0, ROWS_PT)])
        plsc.subcore_barrier()

        sweep(g2_hbm, c * NA)
        plsc.subcore_barrier()

        pltpu.sync_copy(
            acc_sh.at[pl.ds(c * HN + s * (HA // NS), HA // NS)],
            s2_hbm.at[c].at[pl.ds(s * (HA // NS), HA // NS)],
        )

    return deg_pass, gcn_pass


_RB = 1000
_GRID = N_NODES // _RB
_BPH = HN // _RB


def _scale_body(x_ref, w_ref, degp_ref, og_ref, od_ref, ob_ref):
    deg = jnp.sum(degp_ref[...], axis=1) + 1.0
    dinv = lax.rsqrt(deg)
    h = jnp.dot(x_ref[...], w_ref[...], precision=lax.Precision.HIGHEST)
    og_ref[...] = h * dinv[:, None]
    od_ref[...] = dinv[:, None]
    ob_ref[...] = jnp.broadcast_to(dinv[:, None], ob_ref.shape)


def _out_body(s_ref, g2_ref, dinv_ref, w2_ref, b2_ref, o_ref):
    dinv = dinv_ref[...]
    a = (s_ref[0] + g2_ref[...]) * dinv
    o_ref[...] = (
        jnp.dot(a, w2_ref[...], precision=lax.Precision.HIGHEST) + b2_ref[...]
    )


def _row_spec(d):
    return pl.BlockSpec((_RB, d), lambda i: (i, 0))


def _half_spec(d):
    return pl.BlockSpec((1, _RB, d), lambda i: (i // _BPH, i % _BPH, 0))


_deg_spec = pl.BlockSpec((_RB, NW), lambda i: (i, 0))
_vec_spec = pl.BlockSpec((_RB, 1), lambda i: (i, 0))


def _full_spec(shape):
    return pl.BlockSpec(shape, lambda i: tuple(0 for _ in shape))


_scale_call = pl.pallas_call(
    _scale_body,
    grid=(_GRID,),
    in_specs=[
        _row_spec(D_IN),
        _full_spec((D_IN, D_HID)),
        _deg_spec,
    ],
    out_specs=(_row_spec(D_HID), _vec_spec, _row_spec(D_HID)),
    out_shape=(
        jax.ShapeDtypeStruct((NA, D_HID), jnp.float32),
        jax.ShapeDtypeStruct((NA, 1), jnp.float32),
        jax.ShapeDtypeStruct((NA, D_HID), jnp.float32),
    ),
)

_out_call = pl.pallas_call(
    _out_body,
    grid=(_GRID,),
    in_specs=[
        _half_spec(D_HID),
        _row_spec(D_HID),
        _vec_spec,
        _full_spec((D_HID, D_OUT)),
        _full_spec((1, D_OUT)),
    ],
    out_specs=_row_spec(D_OUT),
    out_shape=jax.ShapeDtypeStruct((N_NODES, D_OUT), jnp.float32),
)


def kernel(x, edge_index, W1, b1, W2, b2):
    src = edge_index[0].astype(jnp.int32)
    dst = edge_index[1].astype(jnp.int32)
    pad = E_ALL - N_EDGES
    pad_src = jnp.zeros((pad,), jnp.int32)
    pad_dst = N_NODES + (jnp.arange(pad, dtype=jnp.int32) % N_PAD)
    src2d = jnp.concatenate([src, pad_src]).reshape(NS * RPT_ALL, IB)
    dst2d = jnp.concatenate([dst, pad_dst]).reshape(NS * RPT_ALL, IB)
    zeros_deg = jnp.zeros((NA,), jnp.float32)
    zeros = jnp.zeros((ROWS_PT, D_HID), jnp.float32)

    deg_pass, gcn_pass = _sc_kernels()
    degp = deg_pass(dst2d, zeros_deg).T
    g1, dinv, dinvb = _scale_call(x, W1, degp)
    S2, g2 = gcn_pass(g1, src2d, dst2d, dinvb, b1, zeros)
    return _out_call(S2, g2, dinv, W2, b2.reshape(1, D_OUT))
